# Initial kernel scaffold; baseline (speedup 1.0000x reference)
import jax
import jax.numpy as jnp
from jax.experimental import pallas as pl


def kernel(conf_matrix, src_pcd, tgt_pcd, src_mask, tgt_mask):
    raise NotImplementedError("write your pallas kernel here")


# speedup vs baseline: 28.2510x; 28.2510x over previous
import functools

import jax
import jax.numpy as jnp
import numpy as np
from jax import lax
from jax.experimental import pallas as pl
from jax.experimental.pallas import tpu as pltpu
from jax.experimental.pallas import tpu_sc as plsc

SAMPLE_RATE = 0.2
MAX_CONDITION_NUM = 1000000.0
EPS = 1e-4

NBINS = 4096
NTEC = 32
CAND_CAP = 512
CHUNK = 16384

@functools.cache
def _mesh():
    return plsc.VectorSubcoreMesh(core_axis_name="c", subcore_axis_name="s")


def _wid():
    return lax.axis_index("s") * 2 + lax.axis_index("c")


def _hist_body(per_tec, nchunk, conf_hbm, out_hbm, buf, hist):
    c16 = np.int32(16)
    wid = _wid()
    base = wid * np.int32(per_tec)
    zeros16 = jnp.zeros((16,), jnp.int32)

    def zbody(i, carry):
        hist[pl.ds(i * c16, 16)] = zeros16
        return carry

    lax.fori_loop(0, NBINS * 16 // 16, zbody, 0)

    laneoff = lax.iota(jnp.int32, 16) * np.int32(NBINS)
    ones16 = jnp.ones((16,), jnp.int32)
    scale = jnp.float32(NBINS)

    def chunk_body(ci, carry):
        pltpu.sync_copy(
            conf_hbm.at[pl.ds(base + ci * np.int32(CHUNK), CHUNK)], buf)

        def grp(gi, c2):
            v = buf[pl.ds(gi * c16, 16)]
            bin_ = (v * scale).astype(jnp.int32)
            addr = laneoff + bin_
            plsc.addupdate_scatter(hist, [addr], ones16)
            return c2

        lax.fori_loop(0, CHUNK // 16, grp, 0)
        return carry

    lax.fori_loop(0, nchunk, chunk_body, 0)
    pltpu.sync_copy(hist, out_hbm.at[wid])


def _hist_call(conf_flat, per_tec):
    nchunk = per_tec // CHUNK
    kfn = functools.partial(
        pl.kernel,
        out_type=jax.ShapeDtypeStruct((NTEC, NBINS * 16), jnp.int32),
        mesh=_mesh(),
        scratch_types=[
            pltpu.VMEM((CHUNK,), jnp.float32),
            pltpu.VMEM((NBINS * 16,), jnp.int32),
        ],
        compiler_params=pltpu.CompilerParams(needs_layout_passes=False),
    )(functools.partial(_hist_body, per_tec, nchunk))
    return kfn(conf_flat)


def _thr_body(bsize, k_static, hist_ref, srcm_ref, tgtm_ref, thr_ref):
    h = hist_ref[...]
    h = jnp.sum(h.reshape(NTEC, 16, NBINS), axis=1)
    h8 = jnp.sum(h.reshape(bsize, NTEC // bsize, NBINS), axis=1)

    rc = h8
    s = 1
    while s < NBINS:
        rc = rc + jnp.concatenate(
            [rc[:, s:], jnp.zeros((bsize, s), jnp.int32)], axis=1)
        s *= 2

    srclen = jnp.sum(srcm_ref[...], axis=1)
    tgtlen = jnp.sum(tgtm_ref[...], axis=1)
    entry = jnp.maximum(srclen, tgtlen).astype(jnp.float32) * jnp.float32(
        SAMPLE_RATE)
    k = jnp.minimum(entry.astype(jnp.int32), k_static)

    jidx = lax.broadcasted_iota(jnp.int32, (bsize, NBINS), 1)
    selm = rc >= k[:, None]
    L = jnp.max(jnp.where(selm, jidx, -1), axis=1)
    onehot = jidx == L[:, None]
    nb = jnp.sum(jnp.where(onehot, h8, 0), axis=1)
    rcL = jnp.sum(jnp.where(onehot, rc, 0), axis=1)
    ca = rcL - nb
    need = k - ca

    inv = jnp.float32(1.0 / NBINS)
    u = (L + 1).astype(jnp.float32) * inv
    lo = L.astype(jnp.float32) * inv

    cols = jnp.concatenate(
        [u[:, None], lo[:, None], ca.astype(jnp.float32)[:, None],
         need.astype(jnp.float32)[:, None], k.astype(jnp.float32)[:, None],
         jnp.zeros((bsize, 123), jnp.float32)], axis=1)
    thr_ref[...] = cols


def _thr_call(hists, srcm, tgtm):
    bsize = srcm.shape[0]
    n_total = max(srcm.shape[1], tgtm.shape[1])
    k_static = int(np.int32(np.float32(n_total) * np.float32(SAMPLE_RATE)))
    return pl.pallas_call(
        functools.partial(_thr_body, bsize, k_static),
        out_shape=jax.ShapeDtypeStruct((bsize, 128), jnp.float32),
    )(hists, srcm, tgtm)


def _mom_body(nrb, thr_ref, conf_ref, ye_ref, xa_ref, out_ref):
    b = pl.program_id(0)
    rb = pl.program_id(1)

    @pl.when(rb == 0)
    def _():
        out_ref[...] = jnp.zeros(out_ref.shape, out_ref.dtype)

    u = thr_ref[b, 0]
    w = conf_ref[0]
    wm = jnp.where(w >= u, w, 0.0)
    g = jnp.dot(wm, ye_ref[0], preferred_element_type=jnp.float32)
    out_ref[0] += lax.dot_general(
        xa_ref[0], g, (((0,), (0,)), ((), ())),
        preferred_element_type=jnp.float32)


def _mom_call(thr, conf, ye, xa):
    bsize, n, m = conf.shape
    rblk = min(128, n)
    nrb = n // rblk
    return pl.pallas_call(
        functools.partial(_mom_body, nrb),
        grid=(bsize, nrb),
        in_specs=[
            pl.BlockSpec(memory_space=pltpu.SMEM),
            pl.BlockSpec((1, rblk, m), lambda b, rb: (b, rb, 0)),
            pl.BlockSpec((1, m, 128), lambda b, rb: (b, 0, 0)),
            pl.BlockSpec((1, rblk, 8), lambda b, rb: (b, rb, 0)),
        ],
        out_specs=pl.BlockSpec((1, 8, 128), lambda b, rb: (b, 0, 0)),
        out_shape=jax.ShapeDtypeStruct((bsize, 8, 128), jnp.float32),
    )(thr, conf, ye, xa)


def _cand_body(per_tec, nchunk, nm, mcols, tpb, conf_hbm, srcT_hbm, tgtT_hbm,
               thru_hbm, thrlo_hbm, cand_hbm, cidx_hbm, cnt_hbm,
               buf, x0, x1, x2, y0, y1, y2, cv, cix,
               g0, g1, g2, g3, g4, g5, thru_v, thrlo_v, cnt_v):
    c16 = np.int32(16)
    wid = _wid()
    b = wid // np.int32(tpb)
    base = wid * np.int32(per_tec)
    offq = base - b * np.int32(nm)

    cm = np.int32(mcols)
    b3 = b * np.int32(3)
    pltpu.sync_copy(thru_hbm.at[pl.ds(b * c16, 16)], thru_v)
    pltpu.sync_copy(thrlo_hbm.at[pl.ds(b * c16, 16)], thrlo_v)
    pltpu.sync_copy(srcT_hbm.at[pl.ds((b3 + np.int32(0)) * cm, mcols)], x0)
    pltpu.sync_copy(srcT_hbm.at[pl.ds((b3 + np.int32(1)) * cm, mcols)], x1)
    pltpu.sync_copy(srcT_hbm.at[pl.ds((b3 + np.int32(2)) * cm, mcols)], x2)
    pltpu.sync_copy(tgtT_hbm.at[pl.ds((b3 + np.int32(0)) * cm, mcols)], y0)
    pltpu.sync_copy(tgtT_hbm.at[pl.ds((b3 + np.int32(1)) * cm, mcols)], y1)
    pltpu.sync_copy(tgtT_hbm.at[pl.ds((b3 + np.int32(2)) * cm, mcols)], y2)

    negones = jnp.full((16,), -1.0, jnp.float32)
    zeros16 = jnp.zeros((16,), jnp.int32)

    def ibody(i, carry):
        cv[pl.ds(i * c16, 16)] = negones
        cix[pl.ds(i * c16, 16)] = zeros16
        return carry

    lax.fori_loop(0, CAND_CAP // 16, ibody, 0)

    uvec = thru_v[...]
    lovec = thrlo_v[...]
    lanes = lax.iota(jnp.int32, 16)

    def chunk_body(ci, cnt):
        pltpu.sync_copy(
            conf_hbm.at[pl.ds(base + ci * np.int32(CHUNK), CHUNK)], buf)

        def grp(gi, c2):
            v = buf[pl.ds(gi * c16, 16)]
            m = v >= lovec
            csum = plsc.cumsum(m.astype(jnp.int32))
            pos = jnp.minimum(c2 + csum - np.int32(1), np.int32(CAND_CAP - 1))
            plsc.store_scatter(cv, [pos], v, mask=m)
            iv = (offq + ci * np.int32(CHUNK) + gi * c16) + lanes
            plsc.store_scatter(cix, [pos], iv, mask=m)
            nsel = jnp.max(csum)
            return jnp.minimum(c2 + nsel, np.int32(CAND_CAP))

        return lax.fori_loop(0, CHUNK // 16, grp, cnt)

    cnt = lax.fori_loop(0, nchunk, chunk_body, 0)

    shift = 0
    mm = mcols - 1
    while (1 << shift) < mcols:
        shift += 1

    def gbody(i, carry):
        ii = cix[pl.ds(i * c16, 16)]
        n_i = lax.shift_right_logical(ii, np.int32(shift))
        m_i = jnp.bitwise_and(ii, np.int32(mm))
        g0[pl.ds(i * c16, 16)] = plsc.load_gather(x0, [n_i])
        g1[pl.ds(i * c16, 16)] = plsc.load_gather(x1, [n_i])
        g2[pl.ds(i * c16, 16)] = plsc.load_gather(x2, [n_i])
        g3[pl.ds(i * c16, 16)] = plsc.load_gather(y0, [m_i])
        g4[pl.ds(i * c16, 16)] = plsc.load_gather(y1, [m_i])
        g5[pl.ds(i * c16, 16)] = plsc.load_gather(y2, [m_i])
        return carry

    lax.fori_loop(0, CAND_CAP // 16, gbody, 0)

    cap = np.int32(CAND_CAP)
    ob = wid * np.int32(7 * CAND_CAP)
    pltpu.sync_copy(g0, cand_hbm.at[pl.ds(ob + np.int32(0) * cap, CAND_CAP)])
    pltpu.sync_copy(g1, cand_hbm.at[pl.ds(ob + np.int32(1) * cap, CAND_CAP)])
    pltpu.sync_copy(g2, cand_hbm.at[pl.ds(ob + np.int32(2) * cap, CAND_CAP)])
    pltpu.sync_copy(g3, cand_hbm.at[pl.ds(ob + np.int32(3) * cap, CAND_CAP)])
    pltpu.sync_copy(g4, cand_hbm.at[pl.ds(ob + np.int32(4) * cap, CAND_CAP)])
    pltpu.sync_copy(g5, cand_hbm.at[pl.ds(ob + np.int32(5) * cap, CAND_CAP)])
    pltpu.sync_copy(cv, cand_hbm.at[pl.ds(ob + np.int32(6) * cap, CAND_CAP)])
    pltpu.sync_copy(cix, cidx_hbm.at[pl.ds(wid * cap, CAND_CAP)])
    cnt_v[...] = jnp.zeros((16,), jnp.int32) + cnt
    pltpu.sync_copy(cnt_v, cnt_hbm.at[pl.ds(wid * c16, 16)])


def _cand_call(conf_flat, srcT, tgtT, thru, thrlo, per_tec, nm, mcols, tpb):
    nchunk = per_tec // CHUNK
    kfn = functools.partial(
        pl.kernel,
        out_type=(
            jax.ShapeDtypeStruct((NTEC * 7 * CAND_CAP,), jnp.float32),
            jax.ShapeDtypeStruct((NTEC * CAND_CAP,), jnp.int32),
            jax.ShapeDtypeStruct((NTEC * 16,), jnp.int32),
        ),
        mesh=_mesh(),
        scratch_types=[
            pltpu.VMEM((CHUNK,), jnp.float32),
            pltpu.VMEM((mcols,), jnp.float32),
            pltpu.VMEM((mcols,), jnp.float32),
            pltpu.VMEM((mcols,), jnp.float32),
            pltpu.VMEM((mcols,), jnp.float32),
            pltpu.VMEM((mcols,), jnp.float32),
            pltpu.VMEM((mcols,), jnp.float32),
            pltpu.VMEM((CAND_CAP,), jnp.float32),
            pltpu.VMEM((CAND_CAP,), jnp.int32),
            pltpu.VMEM((CAND_CAP,), jnp.float32),
            pltpu.VMEM((CAND_CAP,), jnp.float32),
            pltpu.VMEM((CAND_CAP,), jnp.float32),
            pltpu.VMEM((CAND_CAP,), jnp.float32),
            pltpu.VMEM((CAND_CAP,), jnp.float32),
            pltpu.VMEM((CAND_CAP,), jnp.float32),
            pltpu.VMEM((16,), jnp.float32),
            pltpu.VMEM((16,), jnp.float32),
            pltpu.VMEM((16,), jnp.int32),
        ],
        compiler_params=pltpu.CompilerParams(needs_layout_passes=False),
    )(functools.partial(_cand_body, per_tec, nchunk, nm, mcols, tpb))
    return kfn(conf_flat, srcT, tgtT, thru, thrlo)


def _dot3(a, b):
    return a[0] * b[0] + a[1] * b[1] + a[2] * b[2]


def _fin_body(ncand, p_ref, i_ref, thr_ref, out_ref):
    b = pl.program_id(0)

    pts = p_ref[0]
    v = pts[6, :]
    valid = pts[7, :] > 0.5
    ii = i_ref[0, 0]
    need = thr_ref[b, 4].astype(jnp.int32)

    va = v[:, None]
    vb = v[None, :]
    ia = ii[:, None]
    ib = ii[None, :]
    beats = jnp.logical_or(vb > va,
                           jnp.logical_and(vb == va, ib < ia))
    beats = jnp.logical_and(beats, valid[None, :])
    rank = jnp.sum(beats.astype(jnp.int32), axis=1)
    sel = jnp.logical_and(valid, rank < need)
    w = jnp.where(sel, v, 0.0)

    x = [pts[0, :], pts[1, :], pts[2, :]]
    y = [pts[3, :], pts[4, :], pts[5, :]]

    s0 = jnp.sum(w)
    wn = w / (s0 + EPS)
    mx = [jnp.sum(wn * x[c]) for c in range(3)]
    my = [jnp.sum(wn * y[a]) for a in range(3)]
    yc = [((y[a] - my[a]) * valid.astype(jnp.float32)).astype(
        jnp.bfloat16).astype(jnp.float32) for a in range(3)]
    pc = [(wn * (x[c] - mx[c])).astype(
        jnp.bfloat16).astype(jnp.float32) for c in range(3)]
    A = [[jnp.sum(yc[a] * pc[c]) for c in range(3)] for a in range(3)]
    sx = [mx[c] for c in range(3)]
    sy = [my[a] for a in range(3)]

    Bc = [[A[0][c], A[1][c], A[2][c]] for c in range(3)]
    Vc = [[jnp.float32(1.0 * (r == c)) for r in range(3)] for c in range(3)]
    for _ in range(8):
        for (p, q) in ((0, 1), (0, 2), (1, 2)):
            app = _dot3(Bc[p], Bc[p])
            aqq = _dot3(Bc[q], Bc[q])
            apq = _dot3(Bc[p], Bc[q])
            denom = 2.0 * apq
            safe = jnp.where(denom == 0.0, 1.0, denom)
            tau = (aqq - app) / safe
            t = jnp.sign(tau) / (jnp.abs(tau) + jnp.sqrt(1.0 + tau * tau))
            t = jnp.where(denom == 0.0, 0.0, t)
            c_ = lax.rsqrt(1.0 + t * t)
            s_ = t * c_
            for r in range(3):
                bp, bq = Bc[p][r], Bc[q][r]
                Bc[p][r] = c_ * bp - s_ * bq
                Bc[q][r] = s_ * bp + c_ * bq
                vp, vq = Vc[p][r], Vc[q][r]
                Vc[p][r] = c_ * vp - s_ * vq
                Vc[q][r] = s_ * vp + c_ * vq

    sig = [jnp.sqrt(_dot3(Bc[c], Bc[c])) for c in range(3)]
    Uc = [[Bc[c][r] / (sig[c] + 1e-30) for r in range(3)] for c in range(3)]

    R0 = [[sum(Uc[c][r] * Vc[c][rv] for c in range(3)) for rv in range(3)]
          for r in range(3)]
    det = (R0[0][0] * (R0[1][1] * R0[2][2] - R0[1][2] * R0[2][1])
           - R0[0][1] * (R0[1][0] * R0[2][2] - R0[1][2] * R0[2][0])
           + R0[0][2] * (R0[1][0] * R0[2][1] - R0[1][1] * R0[2][0]))

    m0 = jnp.logical_and(sig[0] <= sig[1], sig[0] <= sig[2])
    m1 = jnp.logical_and(jnp.logical_not(m0), sig[1] <= sig[2])
    umin = [jnp.where(m0, Uc[0][r], jnp.where(m1, Uc[1][r], Uc[2][r]))
            for r in range(3)]
    vmin = [jnp.where(m0, Vc[0][r], jnp.where(m1, Vc[1][r], Vc[2][r]))
            for r in range(3)]
    flip = det < 0.0
    R = [[jnp.where(flip, R0[r][rv] - 2.0 * umin[r] * vmin[rv], R0[r][rv])
          for rv in range(3)] for r in range(3)]

    smax = jnp.maximum(sig[0], jnp.maximum(sig[1], sig[2]))
    smin = jnp.minimum(sig[0], jnp.minimum(sig[1], sig[2]))
    cond = smax / smin

    t_vec = [my[r] - (R[r][0] * mx[0] + R[r][1] * mx[1] + R[r][2] * mx[2])
             for r in range(3)]

    flat = [R[0][0], R[0][1], R[0][2],
            R[1][0], R[1][1], R[1][2],
            R[2][0], R[2][1], R[2][2],
            t_vec[0], t_vec[1], t_vec[2], cond,
            jnp.sum(sel.astype(jnp.float32)),
            jnp.sum(valid.astype(jnp.float32)),
            need.astype(jnp.float32),
            s0, sx[0], sy[0], 0.0,
            A[0][0], sig[0], sig[1], sig[2], det]
    vals = jnp.stack(flat)
    out_ref[...] = jnp.concatenate(
        [vals, jnp.zeros((103,), jnp.float32)])[None, None, :]


def _fin_call(P, I, thr):
    bsize = P.shape[0]
    ncand = P.shape[2]
    return pl.pallas_call(
        functools.partial(_fin_body, ncand),
        grid=(bsize,),
        in_specs=[
            pl.BlockSpec((1, 8, ncand), lambda b: (b, 0, 0)),
            pl.BlockSpec((1, 1, ncand), lambda b: (b, 0, 0)),
            pl.BlockSpec(memory_space=pltpu.SMEM),
        ],
        out_specs=pl.BlockSpec((1, 1, 128), lambda b: (b, 0, 0)),
        out_shape=jax.ShapeDtypeStruct((bsize, 1, 128), jnp.float32),
    )(P, I.reshape(bsize, 1, ncand), thr)


def kernel(conf_matrix, src_pcd, tgt_pcd, src_mask, tgt_mask):
    with jax.enable_x64(False):
        return _kernel_impl(conf_matrix, src_pcd, tgt_pcd, src_mask, tgt_mask)


def _kernel_impl(conf_matrix, src_pcd, tgt_pcd, src_mask, tgt_mask):
    conf_matrix = conf_matrix.astype(jnp.float32)
    bsize, n, m = conf_matrix.shape
    nm = n * m
    per_tec = bsize * nm // NTEC

    conf_flat = conf_matrix.reshape(bsize * nm)
    hists = _hist_call(conf_flat, per_tec)

    srcm = src_mask.astype(jnp.int32)
    tgtm = tgt_mask.astype(jnp.int32)
    thr = _thr_call(hists, srcm, tgtm)

    src = src_pcd.astype(jnp.float32)
    tgt = tgt_pcd.astype(jnp.float32)
    thru = jnp.broadcast_to(thr[:, 0:1], (bsize, 16)).reshape(bsize * 16)
    thrlo = jnp.broadcast_to(thr[:, 1:2], (bsize, 16)).reshape(bsize * 16)
    srcT = src.transpose(0, 2, 1).reshape(bsize * 3 * n)
    tgtT = tgt.transpose(0, 2, 1).reshape(bsize * 3 * m)
    tpb = NTEC // bsize
    candf, cidxf, cntsf = _cand_call(conf_flat, srcT, tgtT, thru, thrlo,
                                     per_tec, nm, m, tpb)
    cand = candf.reshape(NTEC, 7, CAND_CAP)
    cidx = cidxf.reshape(NTEC, CAND_CAP)
    cnts = cntsf.reshape(NTEC, 16)
    cnt = cnts[:, 0]
    validm = (jnp.arange(CAND_CAP, dtype=jnp.int32)[None, :]
              < cnt[:, None]).astype(jnp.float32)
    ncand = tpb * CAND_CAP
    pb = cand.reshape(bsize, tpb, 7, CAND_CAP).transpose(0, 2, 1, 3).reshape(
        bsize, 7, ncand)
    vrow = validm.reshape(bsize, tpb, CAND_CAP).reshape(bsize, 1, ncand)
    P = jnp.concatenate([pb, vrow], axis=1)
    I = cidx.reshape(bsize, ncand)

    out = _fin_call(P, I, thr).reshape(bsize, 128)

    R = out[:, 0:9].reshape(bsize, 3, 3)
    t = out[:, 9:12].reshape(bsize, 3, 1)
    condition = out[:, 12]
    solution_mask = condition < MAX_CONDITION_NUM
    eye = jnp.eye(3, dtype=R.dtype)[None]
    zt = jnp.zeros((1, 3, 1), dtype=t.dtype)
    R_forwd = jnp.where(solution_mask[:, None, None], R, eye)
    t_forwd = jnp.where(solution_mask[:, None, None], t, zt)
    return (R, t, R_forwd, t_forwd, condition, solution_mask)


# speedup vs baseline: 28.9034x; 1.0231x over previous
import functools

import jax
import jax.numpy as jnp
import numpy as np
from jax import lax
from jax.experimental import pallas as pl
from jax.experimental.pallas import tpu as pltpu
from jax.experimental.pallas import tpu_sc as plsc

SAMPLE_RATE = 0.2
MAX_CONDITION_NUM = 1000000.0
EPS = 1e-4

NBINS = 4096
NTEC = 32
CAND_CAP = 512
CHUNK = 16384

@functools.cache
def _mesh():
    return plsc.VectorSubcoreMesh(core_axis_name="c", subcore_axis_name="s")


def _wid():
    return lax.axis_index("s") * 2 + lax.axis_index("c")


def _hist_body(per_tec, nchunk, conf_hbm, out_hbm, buf, hist):
    c16 = np.int32(16)
    wid = _wid()
    base = wid * np.int32(per_tec)
    zeros16 = jnp.zeros((16,), jnp.int32)

    def zbody(i, carry):
        hist[pl.ds(i * c16, 16)] = zeros16
        return carry

    lax.fori_loop(0, NBINS * 16 // 16, zbody, 0)

    laneoff = lax.iota(jnp.int32, 16) * np.int32(NBINS)
    ones16 = jnp.ones((16,), jnp.int32)
    scale = jnp.float32(NBINS)

    def chunk_body(ci, carry):
        pltpu.sync_copy(
            conf_hbm.at[pl.ds(base + ci * np.int32(CHUNK), CHUNK)], buf)

        def grp(gi, c2):
            for j in range(4):
                v = buf[pl.ds(gi * np.int32(64) + np.int32(j * 16), 16)]
                bin_ = (v * scale).astype(jnp.int32)
                addr = laneoff + bin_
                plsc.addupdate_scatter(hist, [addr], ones16)
            return c2

        lax.fori_loop(0, CHUNK // 64, grp, 0)
        return carry

    lax.fori_loop(0, nchunk, chunk_body, 0)
    pltpu.sync_copy(hist, out_hbm.at[wid])


def _hist_call(conf_flat, per_tec):
    nchunk = per_tec // CHUNK
    kfn = functools.partial(
        pl.kernel,
        out_type=jax.ShapeDtypeStruct((NTEC, NBINS * 16), jnp.int32),
        mesh=_mesh(),
        scratch_types=[
            pltpu.VMEM((CHUNK,), jnp.float32),
            pltpu.VMEM((NBINS * 16,), jnp.int32),
        ],
        compiler_params=pltpu.CompilerParams(needs_layout_passes=False),
    )(functools.partial(_hist_body, per_tec, nchunk))
    return kfn(conf_flat)


def _thr_body(bsize, k_static, hist_ref, srcm_ref, tgtm_ref, thr_ref):
    h = hist_ref[...]
    h = jnp.sum(h.reshape(NTEC, 16, NBINS), axis=1)
    h8 = jnp.sum(h.reshape(bsize, NTEC // bsize, NBINS), axis=1)

    rc = h8
    s = 1
    while s < NBINS:
        rc = rc + jnp.concatenate(
            [rc[:, s:], jnp.zeros((bsize, s), jnp.int32)], axis=1)
        s *= 2

    srclen = jnp.sum(srcm_ref[...], axis=1)
    tgtlen = jnp.sum(tgtm_ref[...], axis=1)
    entry = jnp.maximum(srclen, tgtlen).astype(jnp.float32) * jnp.float32(
        SAMPLE_RATE)
    k = jnp.minimum(entry.astype(jnp.int32), k_static)

    jidx = lax.broadcasted_iota(jnp.int32, (bsize, NBINS), 1)
    selm = rc >= k[:, None]
    L = jnp.max(jnp.where(selm, jidx, -1), axis=1)
    onehot = jidx == L[:, None]
    nb = jnp.sum(jnp.where(onehot, h8, 0), axis=1)
    rcL = jnp.sum(jnp.where(onehot, rc, 0), axis=1)
    ca = rcL - nb
    need = k - ca

    inv = jnp.float32(1.0 / NBINS)
    u = (L + 1).astype(jnp.float32) * inv
    lo = L.astype(jnp.float32) * inv

    cols = jnp.concatenate(
        [u[:, None], lo[:, None], ca.astype(jnp.float32)[:, None],
         need.astype(jnp.float32)[:, None], k.astype(jnp.float32)[:, None],
         jnp.zeros((bsize, 123), jnp.float32)], axis=1)
    thr_ref[...] = cols


def _thr_call(hists, srcm, tgtm):
    bsize = srcm.shape[0]
    n_total = max(srcm.shape[1], tgtm.shape[1])
    k_static = int(np.int32(np.float32(n_total) * np.float32(SAMPLE_RATE)))
    return pl.pallas_call(
        functools.partial(_thr_body, bsize, k_static),
        out_shape=jax.ShapeDtypeStruct((bsize, 128), jnp.float32),
    )(hists, srcm, tgtm)


def _cand_body(per_tec, nchunk, nm, mcols, tpb, conf_hbm, srcT_hbm, tgtT_hbm,
               thru_hbm, thrlo_hbm, cand_hbm, cidx_hbm, cnt_hbm,
               buf, x0, x1, x2, y0, y1, y2, cv, cix,
               g0, g1, g2, g3, g4, g5, thru_v, thrlo_v, cnt_v):
    c16 = np.int32(16)
    wid = _wid()
    b = wid // np.int32(tpb)
    base = wid * np.int32(per_tec)
    offq = base - b * np.int32(nm)

    cm = np.int32(mcols)
    b3 = b * np.int32(3)
    pltpu.sync_copy(thru_hbm.at[pl.ds(b * c16, 16)], thru_v)
    pltpu.sync_copy(thrlo_hbm.at[pl.ds(b * c16, 16)], thrlo_v)
    pltpu.sync_copy(srcT_hbm.at[pl.ds((b3 + np.int32(0)) * cm, mcols)], x0)
    pltpu.sync_copy(srcT_hbm.at[pl.ds((b3 + np.int32(1)) * cm, mcols)], x1)
    pltpu.sync_copy(srcT_hbm.at[pl.ds((b3 + np.int32(2)) * cm, mcols)], x2)
    pltpu.sync_copy(tgtT_hbm.at[pl.ds((b3 + np.int32(0)) * cm, mcols)], y0)
    pltpu.sync_copy(tgtT_hbm.at[pl.ds((b3 + np.int32(1)) * cm, mcols)], y1)
    pltpu.sync_copy(tgtT_hbm.at[pl.ds((b3 + np.int32(2)) * cm, mcols)], y2)

    negones = jnp.full((16,), -1.0, jnp.float32)
    zeros16 = jnp.zeros((16,), jnp.int32)

    def ibody(i, carry):
        cv[pl.ds(i * c16, 16)] = negones
        cix[pl.ds(i * c16, 16)] = zeros16
        return carry

    lax.fori_loop(0, CAND_CAP // 16, ibody, 0)

    uvec = thru_v[...]
    lovec = thrlo_v[...]
    lanes = lax.iota(jnp.int32, 16)

    def chunk_body(ci, cnt):
        pltpu.sync_copy(
            conf_hbm.at[pl.ds(base + ci * np.int32(CHUNK), CHUNK)], buf)

        def grp(gi, c2):
            for j in range(4):
                off = gi * np.int32(64) + np.int32(j * 16)
                v = buf[pl.ds(off, 16)]
                m = v >= lovec
                csum = plsc.cumsum(m.astype(jnp.int32))
                pos = jnp.minimum(c2 + csum - np.int32(1),
                                  np.int32(CAND_CAP - 1))
                plsc.store_scatter(cv, [pos], v, mask=m)
                iv = (offq + ci * np.int32(CHUNK) + off) + lanes
                plsc.store_scatter(cix, [pos], iv, mask=m)
                nsel = jnp.max(csum)
                c2 = jnp.minimum(c2 + nsel, np.int32(CAND_CAP))
            return c2

        return lax.fori_loop(0, CHUNK // 64, grp, cnt)

    cnt = lax.fori_loop(0, nchunk, chunk_body, 0)

    shift = 0
    mm = mcols - 1
    while (1 << shift) < mcols:
        shift += 1

    def gbody(i, carry):
        ii = cix[pl.ds(i * c16, 16)]
        n_i = lax.shift_right_logical(ii, np.int32(shift))
        m_i = jnp.bitwise_and(ii, np.int32(mm))
        g0[pl.ds(i * c16, 16)] = plsc.load_gather(x0, [n_i])
        g1[pl.ds(i * c16, 16)] = plsc.load_gather(x1, [n_i])
        g2[pl.ds(i * c16, 16)] = plsc.load_gather(x2, [n_i])
        g3[pl.ds(i * c16, 16)] = plsc.load_gather(y0, [m_i])
        g4[pl.ds(i * c16, 16)] = plsc.load_gather(y1, [m_i])
        g5[pl.ds(i * c16, 16)] = plsc.load_gather(y2, [m_i])
        return carry

    lax.fori_loop(0, CAND_CAP // 16, gbody, 0)

    cap = np.int32(CAND_CAP)
    ob = wid * np.int32(7 * CAND_CAP)
    pltpu.sync_copy(g0, cand_hbm.at[pl.ds(ob + np.int32(0) * cap, CAND_CAP)])
    pltpu.sync_copy(g1, cand_hbm.at[pl.ds(ob + np.int32(1) * cap, CAND_CAP)])
    pltpu.sync_copy(g2, cand_hbm.at[pl.ds(ob + np.int32(2) * cap, CAND_CAP)])
    pltpu.sync_copy(g3, cand_hbm.at[pl.ds(ob + np.int32(3) * cap, CAND_CAP)])
    pltpu.sync_copy(g4, cand_hbm.at[pl.ds(ob + np.int32(4) * cap, CAND_CAP)])
    pltpu.sync_copy(g5, cand_hbm.at[pl.ds(ob + np.int32(5) * cap, CAND_CAP)])
    pltpu.sync_copy(cv, cand_hbm.at[pl.ds(ob + np.int32(6) * cap, CAND_CAP)])
    pltpu.sync_copy(cix, cidx_hbm.at[pl.ds(wid * cap, CAND_CAP)])
    cnt_v[...] = jnp.zeros((16,), jnp.int32) + cnt
    pltpu.sync_copy(cnt_v, cnt_hbm.at[pl.ds(wid * c16, 16)])


def _cand_call(conf_flat, srcT, tgtT, thru, thrlo, per_tec, nm, mcols, tpb):
    nchunk = per_tec // CHUNK
    kfn = functools.partial(
        pl.kernel,
        out_type=(
            jax.ShapeDtypeStruct((NTEC * 7 * CAND_CAP,), jnp.float32),
            jax.ShapeDtypeStruct((NTEC * CAND_CAP,), jnp.int32),
            jax.ShapeDtypeStruct((NTEC * 16,), jnp.int32),
        ),
        mesh=_mesh(),
        scratch_types=[
            pltpu.VMEM((CHUNK,), jnp.float32),
            pltpu.VMEM((mcols,), jnp.float32),
            pltpu.VMEM((mcols,), jnp.float32),
            pltpu.VMEM((mcols,), jnp.float32),
            pltpu.VMEM((mcols,), jnp.float32),
            pltpu.VMEM((mcols,), jnp.float32),
            pltpu.VMEM((mcols,), jnp.float32),
            pltpu.VMEM((CAND_CAP,), jnp.float32),
            pltpu.VMEM((CAND_CAP,), jnp.int32),
            pltpu.VMEM((CAND_CAP,), jnp.float32),
            pltpu.VMEM((CAND_CAP,), jnp.float32),
            pltpu.VMEM((CAND_CAP,), jnp.float32),
            pltpu.VMEM((CAND_CAP,), jnp.float32),
            pltpu.VMEM((CAND_CAP,), jnp.float32),
            pltpu.VMEM((CAND_CAP,), jnp.float32),
            pltpu.VMEM((16,), jnp.float32),
            pltpu.VMEM((16,), jnp.float32),
            pltpu.VMEM((16,), jnp.int32),
        ],
        compiler_params=pltpu.CompilerParams(needs_layout_passes=False),
    )(functools.partial(_cand_body, per_tec, nchunk, nm, mcols, tpb))
    return kfn(conf_flat, srcT, tgtT, thru, thrlo)


def _dot3(a, b):
    return a[0] * b[0] + a[1] * b[1] + a[2] * b[2]


def _fin_body(ncand, p_ref, i_ref, thr_ref, out_ref):
    b = pl.program_id(0)

    pts = p_ref[0]
    v = pts[6, :]
    valid = pts[7, :] > 0.5
    ii = i_ref[0, 0]
    need = thr_ref[b, 4].astype(jnp.int32)

    va = v[:, None]
    vb = v[None, :]
    ia = ii[:, None]
    ib = ii[None, :]
    beats = jnp.logical_or(vb > va,
                           jnp.logical_and(vb == va, ib < ia))
    beats = jnp.logical_and(beats, valid[None, :])
    rank = jnp.sum(beats.astype(jnp.int32), axis=1)
    sel = jnp.logical_and(valid, rank < need)
    w = jnp.where(sel, v, 0.0)

    x = [pts[0, :], pts[1, :], pts[2, :]]
    y = [pts[3, :], pts[4, :], pts[5, :]]

    s0 = jnp.sum(w)
    wn = w / (s0 + EPS)
    mx = [jnp.sum(wn * x[c]) for c in range(3)]
    my = [jnp.sum(wn * y[a]) for a in range(3)]
    yc = [((y[a] - my[a]) * valid.astype(jnp.float32)).astype(
        jnp.bfloat16).astype(jnp.float32) for a in range(3)]
    pc = [(wn * (x[c] - mx[c])).astype(
        jnp.bfloat16).astype(jnp.float32) for c in range(3)]
    A = [[jnp.sum(yc[a] * pc[c]) for c in range(3)] for a in range(3)]
    sx = [mx[c] for c in range(3)]
    sy = [my[a] for a in range(3)]

    Bc = [[A[0][c], A[1][c], A[2][c]] for c in range(3)]
    Vc = [[jnp.float32(1.0 * (r == c)) for r in range(3)] for c in range(3)]
    for _ in range(8):
        for (p, q) in ((0, 1), (0, 2), (1, 2)):
            app = _dot3(Bc[p], Bc[p])
            aqq = _dot3(Bc[q], Bc[q])
            apq = _dot3(Bc[p], Bc[q])
            denom = 2.0 * apq
            safe = jnp.where(denom == 0.0, 1.0, denom)
            tau = (aqq - app) / safe
            t = jnp.sign(tau) / (jnp.abs(tau) + jnp.sqrt(1.0 + tau * tau))
            t = jnp.where(denom == 0.0, 0.0, t)
            c_ = lax.rsqrt(1.0 + t * t)
            s_ = t * c_
            for r in range(3):
                bp, bq = Bc[p][r], Bc[q][r]
                Bc[p][r] = c_ * bp - s_ * bq
                Bc[q][r] = s_ * bp + c_ * bq
                vp, vq = Vc[p][r], Vc[q][r]
                Vc[p][r] = c_ * vp - s_ * vq
                Vc[q][r] = s_ * vp + c_ * vq

    sig = [jnp.sqrt(_dot3(Bc[c], Bc[c])) for c in range(3)]
    Uc = [[Bc[c][r] / (sig[c] + 1e-30) for r in range(3)] for c in range(3)]

    R0 = [[sum(Uc[c][r] * Vc[c][rv] for c in range(3)) for rv in range(3)]
          for r in range(3)]
    det = (R0[0][0] * (R0[1][1] * R0[2][2] - R0[1][2] * R0[2][1])
           - R0[0][1] * (R0[1][0] * R0[2][2] - R0[1][2] * R0[2][0])
           + R0[0][2] * (R0[1][0] * R0[2][1] - R0[1][1] * R0[2][0]))

    m0 = jnp.logical_and(sig[0] <= sig[1], sig[0] <= sig[2])
    m1 = jnp.logical_and(jnp.logical_not(m0), sig[1] <= sig[2])
    umin = [jnp.where(m0, Uc[0][r], jnp.where(m1, Uc[1][r], Uc[2][r]))
            for r in range(3)]
    vmin = [jnp.where(m0, Vc[0][r], jnp.where(m1, Vc[1][r], Vc[2][r]))
            for r in range(3)]
    flip = det < 0.0
    R = [[jnp.where(flip, R0[r][rv] - 2.0 * umin[r] * vmin[rv], R0[r][rv])
          for rv in range(3)] for r in range(3)]

    smax = jnp.maximum(sig[0], jnp.maximum(sig[1], sig[2]))
    smin = jnp.minimum(sig[0], jnp.minimum(sig[1], sig[2]))
    cond = smax / smin

    t_vec = [my[r] - (R[r][0] * mx[0] + R[r][1] * mx[1] + R[r][2] * mx[2])
             for r in range(3)]

    flat = [R[0][0], R[0][1], R[0][2],
            R[1][0], R[1][1], R[1][2],
            R[2][0], R[2][1], R[2][2],
            t_vec[0], t_vec[1], t_vec[2], cond,
            jnp.sum(sel.astype(jnp.float32)),
            jnp.sum(valid.astype(jnp.float32)),
            need.astype(jnp.float32),
            s0, sx[0], sy[0], 0.0,
            A[0][0], sig[0], sig[1], sig[2], det]
    vals = jnp.stack(flat)
    out_ref[...] = jnp.concatenate(
        [vals, jnp.zeros((103,), jnp.float32)])[None, None, :]


def _fin_call(P, I, thr):
    bsize = P.shape[0]
    ncand = P.shape[2]
    return pl.pallas_call(
        functools.partial(_fin_body, ncand),
        grid=(bsize,),
        in_specs=[
            pl.BlockSpec((1, 8, ncand), lambda b: (b, 0, 0)),
            pl.BlockSpec((1, 1, ncand), lambda b: (b, 0, 0)),
            pl.BlockSpec(memory_space=pltpu.SMEM),
        ],
        out_specs=pl.BlockSpec((1, 1, 128), lambda b: (b, 0, 0)),
        out_shape=jax.ShapeDtypeStruct((bsize, 1, 128), jnp.float32),
    )(P, I.reshape(bsize, 1, ncand), thr)


def kernel(conf_matrix, src_pcd, tgt_pcd, src_mask, tgt_mask):
    with jax.enable_x64(False):
        return _kernel_impl(conf_matrix, src_pcd, tgt_pcd, src_mask, tgt_mask)


def _kernel_impl(conf_matrix, src_pcd, tgt_pcd, src_mask, tgt_mask):
    conf_matrix = conf_matrix.astype(jnp.float32)
    bsize, n, m = conf_matrix.shape
    nm = n * m
    per_tec = bsize * nm // NTEC

    conf_flat = conf_matrix.reshape(bsize * nm)
    hists = _hist_call(conf_flat, per_tec)

    srcm = src_mask.astype(jnp.int32)
    tgtm = tgt_mask.astype(jnp.int32)
    thr = _thr_call(hists, srcm, tgtm)

    src = src_pcd.astype(jnp.float32)
    tgt = tgt_pcd.astype(jnp.float32)
    thru = jnp.broadcast_to(thr[:, 0:1], (bsize, 16)).reshape(bsize * 16)
    thrlo = jnp.broadcast_to(thr[:, 1:2], (bsize, 16)).reshape(bsize * 16)
    srcT = src.transpose(0, 2, 1).reshape(bsize * 3 * n)
    tgtT = tgt.transpose(0, 2, 1).reshape(bsize * 3 * m)
    tpb = NTEC // bsize
    candf, cidxf, cntsf = _cand_call(conf_flat, srcT, tgtT, thru, thrlo,
                                     per_tec, nm, m, tpb)
    cand = candf.reshape(NTEC, 7, CAND_CAP)
    cidx = cidxf.reshape(NTEC, CAND_CAP)
    cnts = cntsf.reshape(NTEC, 16)
    cnt = cnts[:, 0]
    validm = (jnp.arange(CAND_CAP, dtype=jnp.int32)[None, :]
              < cnt[:, None]).astype(jnp.float32)
    ncand = tpb * CAND_CAP
    pb = cand.reshape(bsize, tpb, 7, CAND_CAP).transpose(0, 2, 1, 3).reshape(
        bsize, 7, ncand)
    vrow = validm.reshape(bsize, tpb, CAND_CAP).reshape(bsize, 1, ncand)
    P = jnp.concatenate([pb, vrow], axis=1)
    I = cidx.reshape(bsize, ncand)

    out = _fin_call(P, I, thr).reshape(bsize, 128)

    R = out[:, 0:9].reshape(bsize, 3, 3)
    t = out[:, 9:12].reshape(bsize, 3, 1)
    condition = out[:, 12]
    solution_mask = condition < MAX_CONDITION_NUM
    eye = jnp.eye(3, dtype=R.dtype)[None]
    zt = jnp.zeros((1, 3, 1), dtype=t.dtype)
    R_forwd = jnp.where(solution_mask[:, None, None], R, eye)
    t_forwd = jnp.where(solution_mask[:, None, None], t, zt)
    return (R, t, R_forwd, t_forwd, condition, solution_mask)
